# Initial kernel scaffold; baseline (speedup 1.0000x reference)
import jax
import jax.numpy as jnp
from jax.experimental import pallas as pl


def kernel(x, edge_index, W1l, W1r, b1, W2l, W2r, b2, Wlin, blin):
    raise NotImplementedError("write your pallas kernel here")


# speedup vs baseline: 5.4461x; 5.4461x over previous
import functools

import jax
import jax.numpy as jnp
import numpy as np
from jax import lax
from jax.experimental import pallas as pl
from jax.experimental.pallas import tpu as pltpu
from jax.experimental.pallas import tpu_sc as plsc

NC = 2
NS = 16
CHUNK = 80
LANES = 128


def _make_sc_segsum(n, e, w, with_counts):
    nw = NC * NS
    per_w = e // nw
    assert per_w * nw == e and per_w % CHUNK == 0
    n_chunks = per_w // CHUNK
    rpt = -(-n // (8 * NS)) * 8
    n_pad = rpt * NS
    cr = -(-(-(-n // LANES)) // 8) * 8
    assert cr <= rpt
    mesh = plsc.VectorSubcoreMesh(core_axis_name="c", subcore_axis_name="s")

    def body(table, src, dst, zeros, iota, out, *rest):
        if with_counts:
            (cnt_out, src_v, dst_v, rows_v, cnt_v, iota_v,
             acc, cnt_sh, sem) = rest
        else:
            src_v, dst_v, rows_v, acc, sem = rest
        c = lax.axis_index("c")
        s = lax.axis_index("s")
        g = s * NC + c
        pltpu.sync_copy(zeros, acc.at[pl.ds(s * rpt, rpt)])
        if with_counts:
            pltpu.sync_copy(zeros.at[pl.ds(0, cr)], cnt_v)

            @pl.when(s == 0)
            def _():
                pltpu.sync_copy(zeros.at[pl.ds(0, cr)], cnt_sh)
        plsc.subcore_barrier()

        def chunk(i, carry):
            base = g * per_w + i * CHUNK
            pltpu.sync_copy(src.at[pl.ds(base, CHUNK)], src_v)
            pltpu.sync_copy(dst.at[pl.ds(base, CHUNK)], dst_v)
            pltpu.async_copy(table.at[src_v], rows_v, sem).wait()
            pltpu.sync_copy(rows_v, acc.at[dst_v], add=True)
            if with_counts:
                ones = jnp.ones((16,), jnp.float32)
                for k in range(CHUNK // 16):
                    d16 = dst_v[pl.ds(k * 16, 16)]
                    row = lax.shift_right_logical(d16, 7)
                    col = lax.bitwise_and(d16, 127)
                    plsc.addupdate_scatter(cnt_v, [row, col], ones)
            return carry

        lax.fori_loop(0, n_chunks, chunk, 0)
        plsc.subcore_barrier()
        if with_counts:
            pltpu.sync_copy(iota, iota_v)
            pltpu.sync_copy(cnt_v, cnt_sh.at[iota_v], add=True)
        plsc.subcore_barrier()
        pltpu.sync_copy(acc.at[pl.ds(s * rpt, rpt)],
                        out.at[pl.ds(c * n_pad + s * rpt, rpt)])
        if with_counts:
            @pl.when(s == 0)
            def _():
                pltpu.sync_copy(cnt_sh, cnt_out.at[c])

    out_type = [jax.ShapeDtypeStruct((NC * n_pad, w), jnp.float32)]
    scratch = [
        pltpu.VMEM((CHUNK,), jnp.int32),
        pltpu.VMEM((CHUNK,), jnp.int32),
        pltpu.VMEM((CHUNK, w), jnp.float32),
        pltpu.VMEM_SHARED((n_pad, w), jnp.float32),
        pltpu.SemaphoreType.DMA,
    ]
    if with_counts:
        out_type.append(jax.ShapeDtypeStruct((NC, cr, LANES), jnp.float32))
        scratch.insert(3, pltpu.VMEM((cr, LANES), jnp.float32))
        scratch.insert(4, pltpu.VMEM((cr,), jnp.int32))
        scratch.insert(6, pltpu.VMEM_SHARED((cr, LANES), jnp.float32))
    return pl.kernel(
        body, out_type=out_type, mesh=mesh, scratch_types=scratch,
        compiler_params=pltpu.CompilerParams(needs_layout_passes=False),
    ), n_pad, cr


def _dotT(a, b):
    return lax.dot_general(a, b, (((1,), (1,)), ((), ())),
                           preferred_element_type=jnp.float32)


def _layer1_body(p_ref, cnt_ref, x_ref, wl_ref, wr_ref, b_ref,
                 h_ref, rcp_ref):
    cnt = cnt_ref[0] + cnt_ref[1]
    rcp = 1.0 / jnp.maximum(cnt, 1.0)
    mean = (p_ref[0] + p_ref[1]) * rcp
    h = _dotT(mean, wl_ref[...]) + _dotT(x_ref[...], wr_ref[...]) + b_ref[...]
    h_ref[...] = jnp.maximum(h, 0.0)
    rcp_ref[...] = rcp


def _layer2_body(p_ref, h_ref, rcp_ref, wl_ref, wr_ref, b_ref,
                 wlin_ref, blin_ref, o_ref):
    mean = (p_ref[0] + p_ref[1]) * rcp_ref[...]
    z = _dotT(mean, wl_ref[...]) + _dotT(h_ref[...], wr_ref[...]) + b_ref[...]
    z = jnp.maximum(z, 0.0)
    o_ref[...] = (jnp.sum(z * wlin_ref[...], axis=1, keepdims=True)
                  + blin_ref[0, 0])


RB = 1000


def kernel(x, edge_index, W1l, W1r, b1, W2l, W2r, b2, Wlin, blin):
    n, d = x.shape
    e = edge_index.shape[1]
    h_dim = W1l.shape[0]
    src = edge_index[0]
    dst = edge_index[1]

    seg1, n_pad, cr = _make_sc_segsum(n, e, d, with_counts=True)
    zeros = jnp.zeros((n_pad // NS, d), jnp.float32)
    iota = jnp.arange(cr, dtype=jnp.int32)
    p1, cnts = seg1(x, src, dst, zeros, iota)
    p1 = p1.reshape(NC, n_pad, d)
    cnts = cnts.reshape(NC, cr * LANES, 1)

    grid = n // RB
    assert grid * RB == n
    h, rcp = pl.pallas_call(
        _layer1_body,
        grid=(grid,),
        in_specs=[
            pl.BlockSpec((NC, RB, d), lambda i: (0, i, 0)),
            pl.BlockSpec((NC, RB, 1), lambda i: (0, i, 0)),
            pl.BlockSpec((RB, d), lambda i: (i, 0)),
            pl.BlockSpec((h_dim, d), lambda i: (0, 0)),
            pl.BlockSpec((h_dim, d), lambda i: (0, 0)),
            pl.BlockSpec((1, h_dim), lambda i: (0, 0)),
        ],
        out_specs=[
            pl.BlockSpec((RB, h_dim), lambda i: (i, 0)),
            pl.BlockSpec((RB, 1), lambda i: (i, 0)),
        ],
        out_shape=[
            jax.ShapeDtypeStruct((n, h_dim), jnp.float32),
            jax.ShapeDtypeStruct((n, 1), jnp.float32),
        ],
    )(p1, cnts, x, W1l, W1r, b1.reshape(1, -1))

    seg2, n_pad2, _ = _make_sc_segsum(n, e, h_dim, with_counts=False)
    zeros2 = jnp.zeros((n_pad2 // NS, h_dim), jnp.float32)
    (p2,) = seg2(h, src, dst, zeros2, iota)
    p2 = p2.reshape(NC, n_pad2, h_dim)

    out = pl.pallas_call(
        _layer2_body,
        grid=(grid,),
        in_specs=[
            pl.BlockSpec((NC, RB, h_dim), lambda i: (0, i, 0)),
            pl.BlockSpec((RB, h_dim), lambda i: (i, 0)),
            pl.BlockSpec((RB, 1), lambda i: (i, 0)),
            pl.BlockSpec((h_dim, h_dim), lambda i: (0, 0)),
            pl.BlockSpec((h_dim, h_dim), lambda i: (0, 0)),
            pl.BlockSpec((1, h_dim), lambda i: (0, 0)),
            pl.BlockSpec((1, h_dim), lambda i: (0, 0)),
            pl.BlockSpec((1, 1), lambda i: (0, 0)),
        ],
        out_specs=pl.BlockSpec((RB, 1), lambda i: (i, 0)),
        out_shape=jax.ShapeDtypeStruct((n, 1), jnp.float32),
    )(p2, h, rcp, W2l, W2r, b2.reshape(1, -1), Wlin, blin.reshape(1, 1))
    return out


# speedup vs baseline: 10.5929x; 1.9450x over previous
import functools

import jax
import jax.numpy as jnp
import numpy as np
from jax import lax
from jax.experimental import pallas as pl
from jax.experimental.pallas import tpu as pltpu
from jax.experimental.pallas import tpu_sc as plsc

NC = 2
NS = 16
CHUNK = 80
PACK = 16384


def _make_sc_segsum(n, e, w):
    nw = NC * NS
    per_w = e // nw
    assert per_w * nw == e and per_w % CHUNK == 0
    nch = per_w // CHUNK
    assert nch >= 3 and nch % 2 == 1
    rpt = -(-n // (8 * NS)) * 8
    n_pad = rpt * NS
    mesh = plsc.VectorSubcoreMesh(core_axis_name="c", subcore_axis_name="s")

    def body(table, pk, zeros, out, pk_v, sbuf0, sbuf1, dbuf0, dbuf1,
             rows0, rows1, acc, gsem, ssem):
        c = lax.axis_index("c")
        s = lax.axis_index("s")
        g = s * NC + c
        pltpu.sync_copy(zeros, acc.at[pl.ds(s * rpt, rpt)])
        pltpu.sync_copy(pk.at[g], pk_v)
        plsc.subcore_barrier()

        dummy = table.at[pl.ds(0, CHUNK)]
        sbufs = (sbuf0, sbuf1)
        dbufs = (dbuf0, dbuf1)
        rows = (rows0, rows1)

        def unpack(j, slot):
            row = pk_v.at[j]
            for k in range(CHUNK // 16):
                p16 = row[pl.ds(k * 16, 16)]
                sbufs[slot][pl.ds(k * 16, 16)] = lax.bitwise_and(
                    p16, PACK - 1)
                dbufs[slot][pl.ds(k * 16, 16)] = lax.shift_right_logical(
                    p16, 14)

        def gather(slot):
            pltpu.async_copy(table.at[sbufs[slot]], rows[slot], gsem)

        def scatter(slot):
            pltpu.async_copy(rows[slot], acc.at[dbufs[slot]], ssem,
                             add=True)

        def drain(slot, sem):
            pltpu.make_async_copy(dummy, rows[slot], sem).wait()

        unpack(0, 0)
        gather(0)
        unpack(1, 1)
        gather(1)
        drain(0, gsem)
        scatter(0)

        def pair(t, carry):
            j0 = 2 * t + 2
            drain(0, ssem)
            unpack(j0, 0)
            gather(0)
            drain(1, gsem)
            scatter(1)
            drain(1, ssem)
            unpack(j0 + 1, 1)
            gather(1)
            drain(0, gsem)
            scatter(0)
            return carry

        lax.fori_loop(0, (nch - 3) // 2, pair, 0)
        drain(0, ssem)
        unpack(nch - 1, 0)
        gather(0)
        drain(1, gsem)
        scatter(1)
        drain(0, gsem)
        scatter(0)
        drain(0, ssem)
        drain(1, ssem)
        plsc.subcore_barrier()
        pltpu.sync_copy(acc.at[pl.ds(s * rpt, rpt)],
                        out.at[pl.ds(c * n_pad + s * rpt, rpt)])

    out_type = [jax.ShapeDtypeStruct((NC * n_pad, w), jnp.float32)]
    scratch = [
        pltpu.VMEM((nch, CHUNK), jnp.int32),
        pltpu.VMEM((CHUNK,), jnp.int32),
        pltpu.VMEM((CHUNK,), jnp.int32),
        pltpu.VMEM((CHUNK,), jnp.int32),
        pltpu.VMEM((CHUNK,), jnp.int32),
        pltpu.VMEM((CHUNK, w), jnp.float32),
        pltpu.VMEM((CHUNK, w), jnp.float32),
        pltpu.VMEM_SHARED((n_pad, w), jnp.float32),
        pltpu.SemaphoreType.DMA,
        pltpu.SemaphoreType.DMA,
    ]
    return pl.kernel(
        body, out_type=out_type, mesh=mesh, scratch_types=scratch,
        compiler_params=pltpu.CompilerParams(
            needs_layout_passes=False, use_tc_tiling_on_sc=False),
    ), n_pad


def _dotT(a, b):
    return lax.dot_general(a, b, (((1,), (1,)), ((), ())),
                           preferred_element_type=jnp.float32)


def _layer1_body(p_ref, x_ref, wl_ref, wr_ref, b_ref, h_ref, rcp_ref):
    p = p_ref[0] + p_ref[1]
    d = x_ref.shape[1]
    cnt = p[:, d:d + 1]
    rcp = 1.0 / jnp.maximum(cnt, 1.0)
    mean = p[:, :d] * rcp
    h = _dotT(mean, wl_ref[...]) + _dotT(x_ref[...], wr_ref[...]) + b_ref[...]
    h_ref[...] = jnp.maximum(h, 0.0)
    rcp_ref[...] = rcp


def _layer2_body(p_ref, h_ref, rcp_ref, wl_ref, wr_ref, b_ref,
                 wlin_ref, blin_ref, o_ref):
    mean = (p_ref[0] + p_ref[1]) * rcp_ref[...]
    z = _dotT(mean, wl_ref[...]) + _dotT(h_ref[...], wr_ref[...]) + b_ref[...]
    z = jnp.maximum(z, 0.0)
    o_ref[...] = (jnp.sum(z * wlin_ref[...], axis=1, keepdims=True)
                  + blin_ref[0, 0])


RB = 1000


def kernel(x, edge_index, W1l, W1r, b1, W2l, W2r, b2, Wlin, blin):
    n, d = x.shape
    e = edge_index.shape[1]
    h_dim = W1l.shape[0]
    nch = e // (NC * NS) // CHUNK
    pk = (edge_index[0] + edge_index[1] * PACK).reshape(NC * NS, nch, CHUNK)

    w1 = d + 8
    xp = jnp.concatenate(
        [x, jnp.ones((n, 1), jnp.float32), jnp.zeros((n, 7), jnp.float32)],
        axis=1)

    seg1, n_pad = _make_sc_segsum(n, e, w1)
    zeros1 = jnp.zeros((n_pad // NS, w1), jnp.float32)
    (p1,) = seg1(xp, pk, zeros1)
    p1 = p1.reshape(NC, n_pad, w1)

    grid = n // RB
    assert grid * RB == n
    h, rcp = pl.pallas_call(
        _layer1_body,
        grid=(grid,),
        in_specs=[
            pl.BlockSpec((NC, RB, w1), lambda i: (0, i, 0)),
            pl.BlockSpec((RB, d), lambda i: (i, 0)),
            pl.BlockSpec((h_dim, d), lambda i: (0, 0)),
            pl.BlockSpec((h_dim, d), lambda i: (0, 0)),
            pl.BlockSpec((1, h_dim), lambda i: (0, 0)),
        ],
        out_specs=[
            pl.BlockSpec((RB, h_dim), lambda i: (i, 0)),
            pl.BlockSpec((RB, 1), lambda i: (i, 0)),
        ],
        out_shape=[
            jax.ShapeDtypeStruct((n, h_dim), jnp.float32),
            jax.ShapeDtypeStruct((n, 1), jnp.float32),
        ],
    )(p1, x, W1l, W1r, b1.reshape(1, -1))

    seg2, n_pad2 = _make_sc_segsum(n, e, h_dim)
    zeros2 = jnp.zeros((n_pad2 // NS, h_dim), jnp.float32)
    (p2,) = seg2(h, pk, zeros2)
    p2 = p2.reshape(NC, n_pad2, h_dim)

    out = pl.pallas_call(
        _layer2_body,
        grid=(grid,),
        in_specs=[
            pl.BlockSpec((NC, RB, h_dim), lambda i: (0, i, 0)),
            pl.BlockSpec((RB, h_dim), lambda i: (i, 0)),
            pl.BlockSpec((RB, 1), lambda i: (i, 0)),
            pl.BlockSpec((h_dim, h_dim), lambda i: (0, 0)),
            pl.BlockSpec((h_dim, h_dim), lambda i: (0, 0)),
            pl.BlockSpec((1, h_dim), lambda i: (0, 0)),
            pl.BlockSpec((1, h_dim), lambda i: (0, 0)),
            pl.BlockSpec((1, 1), lambda i: (0, 0)),
        ],
        out_specs=pl.BlockSpec((RB, 1), lambda i: (i, 0)),
        out_shape=jax.ShapeDtypeStruct((n, 1), jnp.float32),
    )(p2, h, rcp, W2l, W2r, b2.reshape(1, -1), Wlin, blin.reshape(1, 1))
    return out


# speedup vs baseline: 11.8741x; 1.1210x over previous
import functools

import jax
import jax.numpy as jnp
import numpy as np
from jax import lax
from jax.experimental import pallas as pl
from jax.experimental.pallas import tpu as pltpu
from jax.experimental.pallas import tpu_sc as plsc

NC = 2
NS = 16
CHUNK = 80
PACK = 16384


def _make_sc_segsum(n, e, w):
    nw = NC * NS
    per_w = e // nw
    assert per_w * nw == e and per_w % CHUNK == 0
    nch = per_w // CHUNK
    assert nch >= 8 and nch % 3 == 2
    rpt = -(-n // (8 * NS)) * 8
    n_pad = rpt * NS
    mesh = plsc.VectorSubcoreMesh(core_axis_name="c", subcore_axis_name="s")

    def body(table, pk, zeros, out, pk_v,
             sbuf0, sbuf1, sbuf2, dbuf0, dbuf1, dbuf2,
             rows0, rows1, rows2, acc, gsem, ssem):
        c = lax.axis_index("c")
        s = lax.axis_index("s")
        g = s * NC + c
        pltpu.sync_copy(zeros, acc.at[pl.ds(s * rpt, rpt)])
        pltpu.sync_copy(pk.at[g], pk_v)
        plsc.subcore_barrier()

        dummy = table.at[pl.ds(0, CHUNK)]
        sbufs = (sbuf0, sbuf1, sbuf2)
        dbufs = (dbuf0, dbuf1, dbuf2)
        rows = (rows0, rows1, rows2)

        def unpack(j, slot):
            row = pk_v.at[j]
            for k in range(CHUNK // 16):
                p16 = row[pl.ds(k * 16, 16)]
                sbufs[slot][pl.ds(k * 16, 16)] = lax.bitwise_and(
                    p16, PACK - 1)
                dbufs[slot][pl.ds(k * 16, 16)] = lax.shift_right_logical(
                    p16, 14)

        def gather(slot):
            pltpu.async_copy(table.at[sbufs[slot]], rows[slot], gsem)

        def scatter(slot):
            pltpu.async_copy(rows[slot], acc.at[dbufs[slot]], ssem,
                             add=True)

        def drain(slot, sem):
            pltpu.make_async_copy(dummy, rows[slot], sem).wait()

        unpack(0, 0)
        gather(0)
        unpack(1, 1)
        gather(1)
        drain(0, gsem)
        scatter(0)
        unpack(2, 2)
        gather(2)
        drain(1, gsem)
        scatter(1)

        def triple(t, carry):
            for k in range(3):
                j = 3 * t + k
                drain(k, ssem)
                unpack(j, k)
                gather(k)
                drain((k + 2) % 3, gsem)
                scatter((k + 2) % 3)
            return carry

        lax.fori_loop(1, (nch - 2) // 3, triple, 0)
        drain(0, ssem)
        unpack(nch - 2, 0)
        gather(0)
        drain(2, gsem)
        scatter(2)
        drain(1, ssem)
        unpack(nch - 1, 1)
        gather(1)
        drain(0, gsem)
        scatter(0)
        drain(1, gsem)
        scatter(1)
        drain(2, ssem)
        drain(0, ssem)
        drain(1, ssem)
        plsc.subcore_barrier()
        pltpu.sync_copy(acc.at[pl.ds(s * rpt, rpt)],
                        out.at[pl.ds(c * n_pad + s * rpt, rpt)])

    out_type = [jax.ShapeDtypeStruct((NC * n_pad, w), jnp.float32)]
    scratch = (
        [pltpu.VMEM((nch, CHUNK), jnp.int32)]
        + [pltpu.VMEM((CHUNK,), jnp.int32)] * 6
        + [pltpu.VMEM((CHUNK, w), jnp.float32)] * 3
        + [
            pltpu.VMEM_SHARED((n_pad, w), jnp.float32),
            pltpu.SemaphoreType.DMA,
            pltpu.SemaphoreType.DMA,
        ]
    )
    return pl.kernel(
        body, out_type=out_type, mesh=mesh, scratch_types=scratch,
        compiler_params=pltpu.CompilerParams(
            needs_layout_passes=False, use_tc_tiling_on_sc=False),
    ), n_pad


def _dotT(a, b):
    return lax.dot_general(a, b, (((1,), (1,)), ((), ())),
                           preferred_element_type=jnp.float32)


def _layer1_body(p_ref, x_ref, wl_ref, wr_ref, b_ref, h_ref, rcp_ref):
    p = p_ref[0] + p_ref[1]
    d = x_ref.shape[1]
    cnt = p[:, d:d + 1]
    rcp = 1.0 / jnp.maximum(cnt, 1.0)
    mean = p[:, :d] * rcp
    h = _dotT(mean, wl_ref[...]) + _dotT(x_ref[...], wr_ref[...]) + b_ref[...]
    h_ref[...] = jnp.maximum(h, 0.0)
    rcp_ref[...] = rcp


def _layer2_body(p_ref, h_ref, rcp_ref, wl_ref, wr_ref, b_ref,
                 wlin_ref, blin_ref, o_ref):
    mean = (p_ref[0] + p_ref[1]) * rcp_ref[...]
    z = _dotT(mean, wl_ref[...]) + _dotT(h_ref[...], wr_ref[...]) + b_ref[...]
    z = jnp.maximum(z, 0.0)
    o_ref[...] = (jnp.sum(z * wlin_ref[...], axis=1, keepdims=True)
                  + blin_ref[0, 0])


RB = 2000


def kernel(x, edge_index, W1l, W1r, b1, W2l, W2r, b2, Wlin, blin):
    n, d = x.shape
    e = edge_index.shape[1]
    h_dim = W1l.shape[0]
    nch = e // (NC * NS) // CHUNK
    pk = (edge_index[0] + edge_index[1] * PACK).reshape(NC * NS, nch, CHUNK)

    w1 = d + 8
    xp = jnp.concatenate(
        [x, jnp.ones((n, 1), jnp.float32), jnp.zeros((n, 7), jnp.float32)],
        axis=1)

    seg1, n_pad = _make_sc_segsum(n, e, w1)
    zeros1 = jnp.zeros((n_pad // NS, w1), jnp.float32)
    (p1,) = seg1(xp, pk, zeros1)
    p1 = p1.reshape(NC, n_pad, w1)

    grid = n // RB
    assert grid * RB == n
    h, rcp = pl.pallas_call(
        _layer1_body,
        grid=(grid,),
        in_specs=[
            pl.BlockSpec((NC, RB, w1), lambda i: (0, i, 0)),
            pl.BlockSpec((RB, d), lambda i: (i, 0)),
            pl.BlockSpec((h_dim, d), lambda i: (0, 0)),
            pl.BlockSpec((h_dim, d), lambda i: (0, 0)),
            pl.BlockSpec((1, h_dim), lambda i: (0, 0)),
        ],
        out_specs=[
            pl.BlockSpec((RB, h_dim), lambda i: (i, 0)),
            pl.BlockSpec((RB, 1), lambda i: (i, 0)),
        ],
        out_shape=[
            jax.ShapeDtypeStruct((n, h_dim), jnp.float32),
            jax.ShapeDtypeStruct((n, 1), jnp.float32),
        ],
    )(p1, x, W1l, W1r, b1.reshape(1, -1))

    seg2, n_pad2 = _make_sc_segsum(n, e, h_dim)
    zeros2 = jnp.zeros((n_pad2 // NS, h_dim), jnp.float32)
    (p2,) = seg2(h, pk, zeros2)
    p2 = p2.reshape(NC, n_pad2, h_dim)

    out = pl.pallas_call(
        _layer2_body,
        grid=(grid,),
        in_specs=[
            pl.BlockSpec((NC, RB, h_dim), lambda i: (0, i, 0)),
            pl.BlockSpec((RB, h_dim), lambda i: (i, 0)),
            pl.BlockSpec((RB, 1), lambda i: (i, 0)),
            pl.BlockSpec((h_dim, h_dim), lambda i: (0, 0)),
            pl.BlockSpec((h_dim, h_dim), lambda i: (0, 0)),
            pl.BlockSpec((1, h_dim), lambda i: (0, 0)),
            pl.BlockSpec((1, h_dim), lambda i: (0, 0)),
            pl.BlockSpec((1, 1), lambda i: (0, 0)),
        ],
        out_specs=pl.BlockSpec((RB, 1), lambda i: (i, 0)),
        out_shape=jax.ShapeDtypeStruct((n, 1), jnp.float32),
    )(p2, h, rcp, W2l, W2r, b2.reshape(1, -1), Wlin, blin.reshape(1, 1))
    return out
